# baseline (speedup 1.0000x reference)
import jax
import jax.numpy as jnp
from jax import lax
from jax.experimental import pallas as pl
from jax.experimental.pallas import tpu as pltpu
from jax.experimental.pallas import tpu_sc as plsc

BATCH = 16384
EMBED = 32
LANES = 16
PACK = 4
WIDE = EMBED * PACK
CHUNK = 128

_info = plsc.get_sparse_core_info()
_NC = _info.num_cores
_NS = _info.num_subcores
_NW = _NC * _NS
_BPW = BATCH // _NW
_NCHUNK = _BPW // CHUNK


def _sc_body(uid_hbm, iid_hbm, ut_hbm, it_hbm, out_hbm,
             uidx_v, iidx_v, uq_v, iq_v, ubuf, ibuf, out_v,
             sem_u0, sem_u1, sem_i0, sem_i1):
    wid = lax.axis_index("s") * _NC + lax.axis_index("c")
    base = wid * _BPW
    sems_u = (sem_u0, sem_u1)
    sems_i = (sem_i0, sem_i1)

    pltpu.sync_copy(uid_hbm.at[pl.ds(base, _BPW)], uidx_v)
    pltpu.sync_copy(iid_hbm.at[pl.ds(base, _BPW)], iidx_v)

    def to_packed(j, carry):
        sl = pl.ds(j * LANES, LANES)
        uq_v[sl] = lax.shift_right_logical(uidx_v[sl], 2)
        iq_v[sl] = lax.shift_right_logical(iidx_v[sl], 2)
        return carry

    lax.fori_loop(0, _BPW // LANES, to_packed, 0)

    def start(j):
        slot = j % 2
        sl = pl.ds(j * CHUNK, CHUNK)
        cu = pltpu.async_copy(ut_hbm.at[uq_v.at[sl]], ubuf.at[slot], sems_u[slot])
        ci = pltpu.async_copy(it_hbm.at[iq_v.at[sl]], ibuf.at[slot], sems_i[slot])
        return cu, ci

    lane = lax.iota(jnp.int32, LANES)
    inflight = start(0)
    for j in range(_NCHUNK):
        cu, ci = inflight
        nxt = start(j + 1) if j + 1 < _NCHUNK else None
        cu.wait()
        ci.wait()
        slot = j % 2
        for g in range(CHUNK // LANES):
            gsl = pl.ds(j * CHUNK + g * LANES, LANES)
            uoff = jnp.bitwise_and(uidx_v[gsl], PACK - 1) * EMBED
            ioff = jnp.bitwise_and(iidx_v[gsl], PACK - 1) * EMBED
            acc = jnp.zeros((LANES,), jnp.float32)
            for k in range(LANES):
                b = g * LANES + k
                uo = uoff[k]
                io = ioff[k]
                u0 = ubuf[slot, b, pl.ds(uo, LANES)]
                u1 = ubuf[slot, b, pl.ds(uo + LANES, LANES)]
                i0 = ibuf[slot, b, pl.ds(io, LANES)]
                i1 = ibuf[slot, b, pl.ds(io + LANES, LANES)]
                s = jnp.sum(u0 * i0 + u1 * i1)
                acc = jnp.where(lane == k, s, acc)
            out_v[gsl] = acc
        inflight = nxt

    pltpu.sync_copy(out_v, out_hbm.at[pl.ds(base, _BPW)])


@jax.jit
def _impl(user_ids, item_ids, user_table, item_table):
    mesh = plsc.VectorSubcoreMesh(core_axis_name="c", subcore_axis_name="s")
    f = pl.kernel(
        _sc_body,
        out_type=jax.ShapeDtypeStruct((BATCH,), jnp.float32),
        mesh=mesh,
        compiler_params=pltpu.CompilerParams(
            needs_layout_passes=False, use_tc_tiling_on_sc=False),
        scratch_types=[
            pltpu.VMEM((_BPW,), jnp.int32),
            pltpu.VMEM((_BPW,), jnp.int32),
            pltpu.VMEM((_BPW,), jnp.int32),
            pltpu.VMEM((_BPW,), jnp.int32),
            pltpu.VMEM((2, CHUNK, WIDE), jnp.float32),
            pltpu.VMEM((2, CHUNK, WIDE), jnp.float32),
            pltpu.VMEM((_BPW,), jnp.float32),
            pltpu.SemaphoreType.DMA,
            pltpu.SemaphoreType.DMA,
            pltpu.SemaphoreType.DMA,
            pltpu.SemaphoreType.DMA,
        ],
    )
    ut = user_table.reshape(-1, WIDE)
    it = item_table.reshape(-1, WIDE)
    return f(user_ids, item_ids, ut, it)


def kernel(user_ids, item_ids, user_table, item_table):
    return _impl(user_ids.astype(jnp.int32), item_ids.astype(jnp.int32),
                 user_table, item_table)
